# baseline (speedup 1.0000x reference)
import jax
import jax.numpy as jnp
from jax import lax
from jax.experimental import pallas as pl
from jax.experimental.pallas import tpu as pltpu
from jax.experimental.pallas import tpu_sc as plsc

N = 10000
E = 320000
D = 128
HID = 32

NUM_WORKERS = 32
CHUNK = 512
SUB = 128
NSUB = CHUNK // SUB
NGROUPS = CHUNK // 16
NCH = 20
NCHUNKS = NCH * NUM_WORKERS
EP = NCHUNKS * CHUNK
NPAIRS = NCH // 2


def _tc_precompute(x_ref, wa_ref, wb_ref, b1_ref, p_ref, q_ref):
    xv = x_ref[...]
    p_ref[...] = jnp.dot(xv, wa_ref[...], preferred_element_type=jnp.float32)
    q_ref[...] = (jnp.dot(xv, wb_ref[...], preferred_element_type=jnp.float32)
                  + b1_ref[...])


def _sc_edge_mlp(p_hbm, q_hbm, ei_hbm, ef_hbm, w_hbm, out_hbm,
                 oidx_v, didx_v, rows_o, rows_d, ef_v, out_v, w_v,
                 sems, osems):
    wid = lax.axis_index("s") * 2 + lax.axis_index("c")
    pltpu.sync_copy(w_hbm, w_v)
    pltpu.sync_copy(ei_hbm.at[0, wid], oidx_v)
    pltpu.sync_copy(ei_hbm.at[1, wid], didx_v)
    b2vec = jnp.full((16,), w_v[pl.ds(320, 16)][0], jnp.float32)

    def stage(i, cid, buf):
        for j in range(NSUB):
            pltpu.async_copy(p_hbm.at[oidx_v.at[i, j]],
                             rows_o.at[buf, pl.ds(j * SUB, SUB)], sems.at[buf])
            pltpu.async_copy(q_hbm.at[didx_v.at[i, j]],
                             rows_d.at[buf, pl.ds(j * SUB, SUB)], sems.at[buf])
        pltpu.async_copy(ef_hbm.at[cid], ef_v.at[buf], sems.at[buf])

    def drain(buf):
        for j in range(NSUB):
            pltpu.make_async_copy(p_hbm.at[oidx_v.at[0, j]],
                                  rows_o.at[buf, pl.ds(j * SUB, SUB)],
                                  sems.at[buf]).wait()
            pltpu.make_async_copy(q_hbm.at[didx_v.at[0, j]],
                                  rows_d.at[buf, pl.ds(j * SUB, SUB)],
                                  sems.at[buf]).wait()
        pltpu.make_async_copy(ef_hbm.at[0], ef_v.at[buf], sems.at[buf]).wait()

    def compute(k, cid, buf):
        @pl.when(k > 0)
        def _():
            pltpu.make_async_copy(out_v.at[buf],
                                  out_hbm.at[pl.ds(0, CHUNK)],
                                  osems.at[buf]).wait()

        @plsc.parallel_loop(0, NGROUPS, unroll=2)
        def _(g):
            lane = lax.iota(jnp.int32, 16)
            evec = g * 16 + lane
            ef = [plsc.load_gather(
                      ef_v.at[buf], [evec, jnp.full((16,), j, jnp.int32)])
                  for j in range(4)]
            parts = [b2vec,
                     jnp.zeros((16,), jnp.float32),
                     jnp.zeros((16,), jnp.float32),
                     jnp.zeros((16,), jnp.float32)]
            for d in range(HID):
                cvec = (d + lane) & (HID - 1)
                h = (plsc.load_gather(rows_o.at[buf], [evec, cvec])
                     + plsc.load_gather(rows_d.at[buf], [evec, cvec]))
                h = (h + ef[0] * w_v[pl.ds(d, 16)]
                     + ef[1] * w_v[pl.ds(64 + d, 16)]
                     + ef[2] * w_v[pl.ds(128 + d, 16)]
                     + ef[3] * w_v[pl.ds(192 + d, 16)])
                h = jnp.maximum(h, 0.01 * h)
                parts[d % 4] = parts[d % 4] + h * w_v[pl.ds(256 + d, 16)]
            acc = (parts[0] + parts[1]) + (parts[2] + parts[3])
            out_v[buf, pl.ds(g * 16, 16)] = jnp.maximum(acc, 0.01 * acc)

        pltpu.async_copy(out_v.at[buf], out_hbm.at[pl.ds(cid * CHUNK, CHUNK)],
                         osems.at[buf])

    stage(0, wid, 0)

    def pair_body(k, carry):
        i0 = 2 * k
        i1 = 2 * k + 1
        cid0 = wid + NUM_WORKERS * i0
        cid1 = wid + NUM_WORKERS * i1
        stage(i1, cid1, 1)
        drain(0)
        compute(k, cid0, 0)

        i2 = 2 * k + 2

        @pl.when(i2 < NCH)
        def _():
            stage(i2, wid + NUM_WORKERS * i2, 0)

        drain(1)
        compute(k, cid1, 1)
        return carry

    lax.fori_loop(0, NPAIRS, pair_body, 0)
    for buf in range(2):
        pltpu.make_async_copy(out_v.at[buf], out_hbm.at[pl.ds(0, CHUNK)],
                              osems.at[buf]).wait()


def kernel(x, edge_index, edge_features, W1, b1, W2, b2):
    W1a = W1[:D]
    W1b = W1[D:2 * D]
    W1c = W1[2 * D:]
    p, q = pl.pallas_call(
        _tc_precompute,
        out_shape=[jax.ShapeDtypeStruct((N, HID), jnp.float32),
                   jax.ShapeDtypeStruct((N, HID), jnp.float32)],
    )(x, W1a, W1b, b1.reshape(1, HID))

    w2f = W2.reshape(-1)
    wpack = jnp.concatenate(
        [jnp.concatenate([W1c[j], W1c[j]]) for j in range(4)]
        + [w2f, w2f, b2, jnp.zeros((15,), jnp.float32)])

    pad = EP - E
    eip = jnp.concatenate(
        [edge_index, jnp.zeros((2, pad), edge_index.dtype)], axis=1)
    efp = jnp.concatenate(
        [edge_features, jnp.zeros((pad, 4), edge_features.dtype)], axis=0)
    eiw = (eip.reshape(2, NCH, NUM_WORKERS, CHUNK)
           .transpose(0, 2, 1, 3)
           .reshape(2, NUM_WORKERS, NCH, NSUB, SUB))
    ef3 = efp.reshape(NCHUNKS, CHUNK, 4)

    mesh = plsc.VectorSubcoreMesh(core_axis_name="c", subcore_axis_name="s")
    run = pl.kernel(
        _sc_edge_mlp,
        mesh=mesh,
        compiler_params=pltpu.CompilerParams(
            needs_layout_passes=False, use_tc_tiling_on_sc=False),
        out_type=jax.ShapeDtypeStruct((EP,), jnp.float32),
        scratch_types=[
            pltpu.VMEM((NCH, NSUB, SUB), jnp.int32),
            pltpu.VMEM((NCH, NSUB, SUB), jnp.int32),
            pltpu.VMEM((2, CHUNK, HID), jnp.float32),
            pltpu.VMEM((2, CHUNK, HID), jnp.float32),
            pltpu.VMEM((2, CHUNK, 4), jnp.float32),
            pltpu.VMEM((2, CHUNK), jnp.float32),
            pltpu.VMEM((336,), jnp.float32),
            pltpu.SemaphoreType.DMA((2,)),
            pltpu.SemaphoreType.DMA((2,)),
        ],
    )
    outp = run(p, q, eiw, ef3, wpack)
    return outp[:E]


# speedup vs baseline: 1.4140x; 1.4140x over previous
import jax
import jax.numpy as jnp
from jax import lax
from jax.experimental import pallas as pl
from jax.experimental.pallas import tpu as pltpu
from jax.experimental.pallas import tpu_sc as plsc

N = 10000
E = 320000
D = 128
HID = 32

NUM_WORKERS = 32
CHUNK = 512
NCHUNKS = E // CHUNK
SUB = 128
NSUB = CHUNK // SUB
NGROUPS = CHUNK // 16
BASE_CHUNKS = NCHUNKS // NUM_WORKERS
NPAIRS = (BASE_CHUNKS + 2) // 2


def _tc_precompute(x_ref, wa_ref, wb_ref, p_ref, q_ref):
    xv = x_ref[...]
    p_ref[...] = jnp.dot(xv, wa_ref[...], preferred_element_type=jnp.float32)
    q_ref[...] = jnp.dot(xv, wb_ref[...], preferred_element_type=jnp.float32)


EBLK = 8000


def _tc_efc(ef_ref, wc_ref, b1_ref, efc_ref):
    efc_ref[...] = (jnp.dot(ef_ref[...], wc_ref[...],
                            preferred_element_type=jnp.float32) + b1_ref[...])


def _sc_edge_mlp(p_hbm, q_hbm, ei_hbm, efc_hbm, w_hbm, out_hbm,
                 oidx_v, didx_v, rows_o, rows_d, efc_v, out_v, w_v, sems):
    wid = lax.axis_index("s") * 2 + lax.axis_index("c")
    pltpu.sync_copy(w_hbm, w_v)
    b2vec = jnp.full((16,), w_v[pl.ds(64, 16)][0], jnp.float32)

    def stage(cid, buf):
        pltpu.sync_copy(ei_hbm.at[0, cid], oidx_v.at[buf])
        pltpu.sync_copy(ei_hbm.at[1, cid], didx_v.at[buf])
        for j in range(NSUB):
            pltpu.async_copy(p_hbm.at[oidx_v.at[buf, j]],
                             rows_o.at[buf, pl.ds(j * SUB, SUB)], sems.at[buf])
            pltpu.async_copy(q_hbm.at[didx_v.at[buf, j]],
                             rows_d.at[buf, pl.ds(j * SUB, SUB)], sems.at[buf])
        pltpu.async_copy(efc_hbm.at[cid], efc_v.at[buf], sems.at[buf])

    def drain(buf):
        for j in range(NSUB):
            pltpu.make_async_copy(p_hbm.at[oidx_v.at[buf, j]],
                                  rows_o.at[buf, pl.ds(j * SUB, SUB)],
                                  sems.at[buf]).wait()
            pltpu.make_async_copy(q_hbm.at[didx_v.at[buf, j]],
                                  rows_d.at[buf, pl.ds(j * SUB, SUB)],
                                  sems.at[buf]).wait()
        pltpu.make_async_copy(efc_hbm.at[0], efc_v.at[buf], sems.at[buf]).wait()

    def compute(cid, buf):
        base = cid * CHUNK

        @plsc.parallel_loop(0, NGROUPS, unroll=2)
        def _(g):
            lane = lax.iota(jnp.int32, 16)
            evec = g * 16 + lane
            parts = [b2vec,
                     jnp.zeros((16,), jnp.float32),
                     jnp.zeros((16,), jnp.float32),
                     jnp.zeros((16,), jnp.float32)]
            for d in range(HID):
                cvec = (d + lane) & (HID - 1)
                h = (plsc.load_gather(rows_o.at[buf], [evec, cvec])
                     + plsc.load_gather(rows_d.at[buf], [evec, cvec])
                     + plsc.load_gather(efc_v.at[buf], [evec, cvec]))
                h = jnp.maximum(h, 0.01 * h)
                parts[d % 4] = parts[d % 4] + h * w_v[pl.ds(d, 16)]
            acc = (parts[0] + parts[1]) + (parts[2] + parts[3])
            out_v[pl.ds(g * 16, 16)] = jnp.maximum(acc, 0.01 * acc)

        pltpu.sync_copy(out_v, out_hbm.at[pl.ds(base, CHUNK)])

    stage(wid, 0)

    def pair_body(k, carry):
        i1 = 2 * k + 1
        cid1 = wid + NUM_WORKERS * i1

        @pl.when(cid1 < NCHUNKS)
        def _():
            stage(cid1, 1)

        drain(0)
        compute(wid + NUM_WORKERS * 2 * k, 0)

        i2 = 2 * k + 2
        cid2 = wid + NUM_WORKERS * i2

        @pl.when(cid2 < NCHUNKS)
        def _():
            stage(cid2, 0)

        @pl.when(cid1 < NCHUNKS)
        def _():
            drain(1)
            compute(cid1, 1)

        return carry

    lax.fori_loop(0, NPAIRS, pair_body, 0)


def kernel(x, edge_index, edge_features, W1, b1, W2, b2):
    W1a = W1[:D]
    W1b = W1[D:2 * D]
    W1c = W1[2 * D:]
    p, q = pl.pallas_call(
        _tc_precompute,
        out_shape=[jax.ShapeDtypeStruct((N, HID), jnp.float32),
                   jax.ShapeDtypeStruct((N, HID), jnp.float32)],
    )(x, W1a, W1b)

    efc = pl.pallas_call(
        _tc_efc,
        grid=(E // EBLK,),
        in_specs=[pl.BlockSpec((EBLK, 4), lambda i: (i, 0)),
                  pl.BlockSpec((4, HID), lambda i: (0, 0)),
                  pl.BlockSpec((1, HID), lambda i: (0, 0))],
        out_specs=pl.BlockSpec((EBLK, HID), lambda i: (i, 0)),
        out_shape=jax.ShapeDtypeStruct((E, HID), jnp.float32),
    )(edge_features, W1c, b1.reshape(1, HID))

    w2f = W2.reshape(-1)
    wpack = jnp.concatenate([w2f, w2f, b2, jnp.zeros((15,), jnp.float32)])

    ei4 = edge_index.reshape(2, NCHUNKS, NSUB, SUB)
    efc3 = efc.reshape(NCHUNKS, CHUNK, HID)

    mesh = plsc.VectorSubcoreMesh(core_axis_name="c", subcore_axis_name="s")
    run = pl.kernel(
        _sc_edge_mlp,
        mesh=mesh,
        compiler_params=pltpu.CompilerParams(
            needs_layout_passes=False, use_tc_tiling_on_sc=False),
        out_type=jax.ShapeDtypeStruct((E,), jnp.float32),
        scratch_types=[
            pltpu.VMEM((2, NSUB, SUB), jnp.int32),
            pltpu.VMEM((2, NSUB, SUB), jnp.int32),
            pltpu.VMEM((2, CHUNK, HID), jnp.float32),
            pltpu.VMEM((2, CHUNK, HID), jnp.float32),
            pltpu.VMEM((2, CHUNK, HID), jnp.float32),
            pltpu.VMEM((CHUNK,), jnp.float32),
            pltpu.VMEM((80,), jnp.float32),
            pltpu.SemaphoreType.DMA((2,)),
        ],
    )
    return run(p, q, ei4, efc3, wpack)


# speedup vs baseline: 1.6973x; 1.2003x over previous
import jax
import jax.numpy as jnp
from jax import lax
from jax.experimental import pallas as pl
from jax.experimental.pallas import tpu as pltpu
from jax.experimental.pallas import tpu_sc as plsc

N = 10000
E = 320000
D = 128
HID = 32

NUM_WORKERS = 32
CHUNK = 512
NCHUNKS = E // CHUNK
SUB = 128
NSUB = CHUNK // SUB
NGROUPS = CHUNK // 16
BASE_CHUNKS = NCHUNKS // NUM_WORKERS
NPAIRS = (BASE_CHUNKS + 2) // 2


def _tc_precompute(x_ref, wa_ref, wb_ref, p_ref, q_ref):
    xv = x_ref[...]
    p_ref[...] = jnp.dot(xv, wa_ref[...], preferred_element_type=jnp.float32)
    q_ref[...] = jnp.dot(xv, wb_ref[...], preferred_element_type=jnp.float32)


EBLK = 8000


def _tc_efc(ef_ref, wc_ref, b1_ref, efc_ref):
    efc_ref[...] = (jnp.dot(ef_ref[...], wc_ref[...],
                            preferred_element_type=jnp.float32) + b1_ref[...])


def _sc_edge_mlp(p_hbm, q_hbm, ei_hbm, efc_hbm, w_hbm, out_hbm,
                 oidx_v, didx_v, rows_o, rows_d, efc_v, out_v, w_v, sems):
    wid = lax.axis_index("s") * 2 + lax.axis_index("c")
    pltpu.sync_copy(w_hbm, w_v)
    b2vec = jnp.full((16,), w_v[pl.ds(64, 16)][0], jnp.float32)

    def stage(cid, buf):
        pltpu.sync_copy(ei_hbm.at[0, cid], oidx_v.at[buf])
        pltpu.sync_copy(ei_hbm.at[1, cid], didx_v.at[buf])
        for j in range(NSUB):
            pltpu.async_copy(p_hbm.at[oidx_v.at[buf, j]],
                             rows_o.at[buf, pl.ds(j * SUB, SUB)], sems.at[buf])
            pltpu.async_copy(q_hbm.at[didx_v.at[buf, j]],
                             rows_d.at[buf, pl.ds(j * SUB, SUB)], sems.at[buf])
        pltpu.async_copy(efc_hbm.at[cid], efc_v.at[buf], sems.at[buf])

    def drain(buf):
        for j in range(NSUB):
            pltpu.make_async_copy(p_hbm.at[oidx_v.at[buf, j]],
                                  rows_o.at[buf, pl.ds(j * SUB, SUB)],
                                  sems.at[buf]).wait()
            pltpu.make_async_copy(q_hbm.at[didx_v.at[buf, j]],
                                  rows_d.at[buf, pl.ds(j * SUB, SUB)],
                                  sems.at[buf]).wait()
        pltpu.make_async_copy(efc_hbm.at[0], efc_v.at[buf], sems.at[buf]).wait()

    def compute(cid, buf):
        base = cid * CHUNK

        @plsc.parallel_loop(0, NGROUPS, unroll=2)
        def _(g):
            lane = lax.iota(jnp.int32, 16)
            evec = g * 16 + lane
            ebase = evec * HID
            parts = [b2vec,
                     jnp.zeros((16,), jnp.float32),
                     jnp.zeros((16,), jnp.float32),
                     jnp.zeros((16,), jnp.float32)]
            for d in range(HID):
                cvec = (d + lane) & (HID - 1)
                flat = ebase + cvec
                h = (plsc.load_gather(rows_o.at[buf], [evec, cvec])
                     + plsc.load_gather(rows_d.at[buf], [evec, cvec])
                     + plsc.load_gather(
                         efc_v.at[buf],
                         [lax.shift_right_logical(flat, 7), flat & 127]))
                h = jnp.maximum(h, 0.01 * h)
                parts[d % 4] = parts[d % 4] + h * w_v[pl.ds(d, 16)]
            acc = (parts[0] + parts[1]) + (parts[2] + parts[3])
            out_v[pl.ds(g * 16, 16)] = jnp.maximum(acc, 0.01 * acc)

        pltpu.sync_copy(out_v, out_hbm.at[pl.ds(base, CHUNK)])

    stage(wid, 0)

    def pair_body(k, carry):
        i1 = 2 * k + 1
        cid1 = wid + NUM_WORKERS * i1

        @pl.when(cid1 < NCHUNKS)
        def _():
            stage(cid1, 1)

        drain(0)
        compute(wid + NUM_WORKERS * 2 * k, 0)

        i2 = 2 * k + 2
        cid2 = wid + NUM_WORKERS * i2

        @pl.when(cid2 < NCHUNKS)
        def _():
            stage(cid2, 0)

        @pl.when(cid1 < NCHUNKS)
        def _():
            drain(1)
            compute(cid1, 1)

        return carry

    lax.fori_loop(0, NPAIRS, pair_body, 0)


def kernel(x, edge_index, edge_features, W1, b1, W2, b2):
    W1a = W1[:D]
    W1b = W1[D:2 * D]
    W1c = W1[2 * D:]
    p, q = pl.pallas_call(
        _tc_precompute,
        out_shape=[jax.ShapeDtypeStruct((N, HID), jnp.float32),
                   jax.ShapeDtypeStruct((N, HID), jnp.float32)],
    )(x, W1a, W1b)

    ef16 = edge_features.reshape(E // 4, 16)
    wc16 = jnp.einsum("ab,fc->afbc", jnp.eye(4, dtype=W1c.dtype),
                      W1c).reshape(16, 4 * HID)
    b1p = jnp.tile(b1, 4).reshape(1, 4 * HID)
    efc = pl.pallas_call(
        _tc_efc,
        grid=(E // 4 // EBLK,),
        in_specs=[pl.BlockSpec((EBLK, 16), lambda i: (i, 0)),
                  pl.BlockSpec((16, 4 * HID), lambda i: (0, 0)),
                  pl.BlockSpec((1, 4 * HID), lambda i: (0, 0))],
        out_specs=pl.BlockSpec((EBLK, 4 * HID), lambda i: (i, 0)),
        out_shape=jax.ShapeDtypeStruct((E // 4, 4 * HID), jnp.float32),
    )(ef16, wc16, b1p)

    w2f = W2.reshape(-1)
    wpack = jnp.concatenate([w2f, w2f, b2, jnp.zeros((15,), jnp.float32)])

    ei4 = edge_index.reshape(2, NCHUNKS, NSUB, SUB)
    efc3 = efc.reshape(NCHUNKS, SUB, SUB)

    mesh = plsc.VectorSubcoreMesh(core_axis_name="c", subcore_axis_name="s")
    run = pl.kernel(
        _sc_edge_mlp,
        mesh=mesh,
        compiler_params=pltpu.CompilerParams(
            needs_layout_passes=False, use_tc_tiling_on_sc=False),
        out_type=jax.ShapeDtypeStruct((E,), jnp.float32),
        scratch_types=[
            pltpu.VMEM((2, NSUB, SUB), jnp.int32),
            pltpu.VMEM((2, NSUB, SUB), jnp.int32),
            pltpu.VMEM((2, CHUNK, HID), jnp.float32),
            pltpu.VMEM((2, CHUNK, HID), jnp.float32),
            pltpu.VMEM((2, SUB, SUB), jnp.float32),
            pltpu.VMEM((CHUNK,), jnp.float32),
            pltpu.VMEM((80,), jnp.float32),
            pltpu.SemaphoreType.DMA((2,)),
        ],
    )
    return run(p, q, ei4, efc3, wpack)


# speedup vs baseline: 1.8319x; 1.0793x over previous
import jax
import jax.numpy as jnp
from jax import lax
from jax.experimental import pallas as pl
from jax.experimental.pallas import tpu as pltpu
from jax.experimental.pallas import tpu_sc as plsc

N = 10000
E = 320000
D = 128
HID = 32

NUM_WORKERS = 32
CHUNK = 512
NCHUNKS = E // CHUNK
SUB = 128
NSUB = CHUNK // SUB
NGROUPS = CHUNK // 16
BASE_CHUNKS = NCHUNKS // NUM_WORKERS
NPAIRS = (BASE_CHUNKS + 2) // 2


def _tc_precompute(x_ref, wa_ref, wb_ref, p_ref, q_ref):
    xv = x_ref[...]
    p_ref[...] = jnp.dot(xv, wa_ref[...], preferred_element_type=jnp.float32)
    q_ref[...] = jnp.dot(xv, wb_ref[...], preferred_element_type=jnp.float32)


EBLK = 8000


def _tc_efc(ef_ref, wc_ref, b1_ref, efc_ref):
    efc_ref[...] = (jnp.dot(ef_ref[...], wc_ref[...],
                            preferred_element_type=jnp.float32) + b1_ref[...])


def _sc_edge_mlp(p_hbm, q_hbm, ei_hbm, efc_hbm, w_hbm, out_hbm,
                 oidx_v, didx_v, rows_o, rows_d, efc_v, out_v, w_v, sems):
    wid = lax.axis_index("s") * 2 + lax.axis_index("c")
    pltpu.sync_copy(w_hbm, w_v)
    b2vec = jnp.full((16,), w_v[pl.ds(64, 16)][0], jnp.float32)

    def stage(cid, buf):
        pltpu.sync_copy(ei_hbm.at[0, cid], oidx_v.at[buf])
        pltpu.sync_copy(ei_hbm.at[1, cid], didx_v.at[buf])
        for j in range(NSUB):
            pltpu.async_copy(p_hbm.at[oidx_v.at[buf, j]],
                             rows_o.at[buf, pl.ds(j * SUB, SUB)], sems.at[buf])
            pltpu.async_copy(q_hbm.at[didx_v.at[buf, j]],
                             rows_d.at[buf, pl.ds(j * SUB, SUB)], sems.at[buf])
        pltpu.async_copy(efc_hbm.at[cid], efc_v.at[buf], sems.at[buf])

    def drain(buf):
        for j in range(NSUB):
            pltpu.make_async_copy(p_hbm.at[oidx_v.at[buf, j]],
                                  rows_o.at[buf, pl.ds(j * SUB, SUB)],
                                  sems.at[buf]).wait()
            pltpu.make_async_copy(q_hbm.at[didx_v.at[buf, j]],
                                  rows_d.at[buf, pl.ds(j * SUB, SUB)],
                                  sems.at[buf]).wait()
        pltpu.make_async_copy(efc_hbm.at[0], efc_v.at[buf], sems.at[buf]).wait()

    def compute(cid, buf):
        base = cid * CHUNK

        @plsc.parallel_loop(0, NGROUPS, unroll=4)
        def _(g):
            lane = lax.iota(jnp.int32, 16)
            evec = g * 16 + lane
            ebase = evec * HID
            parts = [b2vec,
                     jnp.zeros((16,), jnp.float32),
                     jnp.zeros((16,), jnp.float32),
                     jnp.zeros((16,), jnp.float32)]
            for d in range(HID):
                cvec = (d + lane) & (HID - 1)
                flat = ebase + cvec
                h = (plsc.load_gather(rows_o.at[buf], [evec, cvec])
                     + plsc.load_gather(rows_d.at[buf], [evec, cvec])
                     + plsc.load_gather(
                         efc_v.at[buf],
                         [lax.shift_right_logical(flat, 7), flat & 127]))
                h = jnp.maximum(h, 0.01 * h)
                parts[d % 4] = parts[d % 4] + h * w_v[pl.ds(d, 16)]
            acc = (parts[0] + parts[1]) + (parts[2] + parts[3])
            out_v[pl.ds(g * 16, 16)] = jnp.maximum(acc, 0.01 * acc)

        pltpu.sync_copy(out_v, out_hbm.at[pl.ds(base, CHUNK)])

    stage(wid, 0)

    def pair_body(k, carry):
        i1 = 2 * k + 1
        cid1 = wid + NUM_WORKERS * i1

        @pl.when(cid1 < NCHUNKS)
        def _():
            stage(cid1, 1)

        drain(0)
        compute(wid + NUM_WORKERS * 2 * k, 0)

        i2 = 2 * k + 2
        cid2 = wid + NUM_WORKERS * i2

        @pl.when(cid2 < NCHUNKS)
        def _():
            stage(cid2, 0)

        @pl.when(cid1 < NCHUNKS)
        def _():
            drain(1)
            compute(cid1, 1)

        return carry

    lax.fori_loop(0, NPAIRS, pair_body, 0)


def kernel(x, edge_index, edge_features, W1, b1, W2, b2):
    W1a = W1[:D]
    W1b = W1[D:2 * D]
    W1c = W1[2 * D:]
    p, q = pl.pallas_call(
        _tc_precompute,
        out_shape=[jax.ShapeDtypeStruct((N, HID), jnp.float32),
                   jax.ShapeDtypeStruct((N, HID), jnp.float32)],
    )(x, W1a, W1b)

    ef16 = edge_features.reshape(E // 4, 16)
    wc16 = jnp.einsum("ab,fc->afbc", jnp.eye(4, dtype=W1c.dtype),
                      W1c).reshape(16, 4 * HID)
    b1p = jnp.tile(b1, 4).reshape(1, 4 * HID)
    efc = pl.pallas_call(
        _tc_efc,
        grid=(E // 4 // EBLK,),
        in_specs=[pl.BlockSpec((EBLK, 16), lambda i: (i, 0)),
                  pl.BlockSpec((16, 4 * HID), lambda i: (0, 0)),
                  pl.BlockSpec((1, 4 * HID), lambda i: (0, 0))],
        out_specs=pl.BlockSpec((EBLK, 4 * HID), lambda i: (i, 0)),
        out_shape=jax.ShapeDtypeStruct((E // 4, 4 * HID), jnp.float32),
    )(ef16, wc16, b1p)

    w2f = W2.reshape(-1)
    wpack = jnp.concatenate([w2f, w2f, b2, jnp.zeros((15,), jnp.float32)])

    ei4 = edge_index.reshape(2, NCHUNKS, NSUB, SUB)
    efc3 = efc.reshape(NCHUNKS, SUB, SUB)

    mesh = plsc.VectorSubcoreMesh(core_axis_name="c", subcore_axis_name="s")
    run = pl.kernel(
        _sc_edge_mlp,
        mesh=mesh,
        compiler_params=pltpu.CompilerParams(
            needs_layout_passes=False, use_tc_tiling_on_sc=False),
        out_type=jax.ShapeDtypeStruct((E,), jnp.float32),
        scratch_types=[
            pltpu.VMEM((2, NSUB, SUB), jnp.int32),
            pltpu.VMEM((2, NSUB, SUB), jnp.int32),
            pltpu.VMEM((2, CHUNK, HID), jnp.float32),
            pltpu.VMEM((2, CHUNK, HID), jnp.float32),
            pltpu.VMEM((2, SUB, SUB), jnp.float32),
            pltpu.VMEM((CHUNK,), jnp.float32),
            pltpu.VMEM((80,), jnp.float32),
            pltpu.SemaphoreType.DMA((2,)),
        ],
    )
    return run(p, q, ei4, efc3, wpack)
